# baseline (speedup 1.0000x reference)
import functools

import jax
import jax.numpy as jnp
from jax import lax
from jax.experimental import pallas as pl
from jax.experimental.pallas import tpu as pltpu
from jax.experimental.pallas import tpu_sc as plsc

VOCAB = 100000
DIM = 300
BATCH = 1024
SEQ = 200

_B = BATCH * SEQ
_NW = 32
_BPW = _B // _NW
_C = 200
_NCH = _BPW // _C
_DP = 304


def _make_gather():
    mesh = plsc.VectorSubcoreMesh(core_axis_name="c", subcore_axis_name="s")

    @functools.partial(
        pl.kernel,
        mesh=mesh,
        compiler_params=pltpu.CompilerParams(use_tc_tiling_on_sc=False),
        out_type=jax.ShapeDtypeStruct((_B, _DP), jnp.float32),
        scratch_types=[
            pltpu.VMEM((_C,), jnp.int32),
            pltpu.VMEM((_C, _DP), jnp.float32),
            pltpu.SemaphoreType.DMA,
        ],
    )
    def gather_kernel(idx_hbm, table_hbm, out_hbm, idx_v, rows_v, sem):
        wid = lax.axis_index("s") * 2 + lax.axis_index("c")
        base0 = wid * _BPW

        def body(g, carry):
            base = base0 + g * _C
            pltpu.sync_copy(idx_hbm.at[pl.ds(base, _C)], idx_v)
            pltpu.async_copy(table_hbm.at[idx_v], rows_v, sem).wait()
            pltpu.sync_copy(rows_v, out_hbm.at[pl.ds(base, _C)])
            return carry

        lax.fori_loop(0, _NCH, body, 0)

    return gather_kernel


_gather = _make_gather()


def kernel(word_ids, emb_weight):
    flat_idx = word_ids.reshape(_B)
    table = jnp.pad(emb_weight, ((0, 0), (0, _DP - DIM)))
    out = _gather(flat_idx, table)
    return out[:, :DIM].reshape(BATCH, SEQ, DIM)


# speedup vs baseline: 1.2820x; 1.2820x over previous
import functools

import jax
import jax.numpy as jnp
from jax import lax
from jax.experimental import pallas as pl
from jax.experimental.pallas import tpu as pltpu
from jax.experimental.pallas import tpu_sc as plsc

VOCAB = 100000
DIM = 300
BATCH = 1024
SEQ = 200

_NW = 32
_NBANDS = SEQ // 8
_DPW = DIM // _NW
_EXTRA = DIM - _DPW * _NW


def _make_gather():
    mesh = plsc.VectorSubcoreMesh(core_axis_name="c", subcore_axis_name="s")

    @functools.partial(
        pl.kernel,
        mesh=mesh,
        compiler_params=pltpu.CompilerParams(needs_layout_passes=False),
        out_type=jax.ShapeDtypeStruct((DIM, SEQ, BATCH), jnp.float32),
        scratch_types=[
            pltpu.VMEM((VOCAB,), jnp.float32),
            pltpu.VMEM((8, BATCH), jnp.int32),
            pltpu.VMEM((8, BATCH), jnp.float32),
        ],
    )
    def gather_kernel(ids_hbm, table_hbm, out_hbm, row_v, ids_v, out_v):
        wid = lax.axis_index("s") * 2 + lax.axis_index("c")
        d_start = _DPW * wid + jnp.minimum(wid, _EXTRA)
        d_count = _DPW + jnp.where(wid < _EXTRA, 1, 0)

        def plane_body(k, carry):
            d = d_start + k
            pltpu.sync_copy(table_hbm.at[d], row_v)

            def band_body(band, carry2):
                pltpu.sync_copy(ids_hbm.at[pl.ds(band * 8, 8)], ids_v)

                def vec_body(v, carry3):
                    for j in range(16):
                        n = v * 16 + j
                        r = n // 64
                        c = (n % 64) * 16
                        iv = ids_v[r, pl.ds(c, 16)]
                        out_v[r, pl.ds(c, 16)] = plsc.load_gather(row_v, [iv])
                    return carry3

                lax.fori_loop(0, 32, vec_body, 0)
                pltpu.sync_copy(out_v, out_hbm.at[d, pl.ds(band * 8, 8)])
                return carry2

            lax.fori_loop(0, _NBANDS, band_body, 0)
            return carry

        lax.fori_loop(0, d_count, plane_body, 0)

    return gather_kernel


_gather = _make_gather()


def kernel(word_ids, emb_weight):
    out_p = _gather(word_ids.T, emb_weight.T)
    return out_p.transpose(2, 1, 0)


# speedup vs baseline: 2.4418x; 1.9046x over previous
import functools

import jax
import jax.numpy as jnp
from jax import lax
from jax.experimental import pallas as pl
from jax.experimental.pallas import tpu as pltpu
from jax.experimental.pallas import tpu_sc as plsc

VOCAB = 100000
DIM = 300
BATCH = 1024
SEQ = 200

_NW = 32
_NBANDS = SEQ // 8
_DPW = DIM // _NW
_EXTRA = DIM - _DPW * _NW


def _make_gather():
    mesh = plsc.VectorSubcoreMesh(core_axis_name="c", subcore_axis_name="s")

    @functools.partial(
        pl.kernel,
        mesh=mesh,
        compiler_params=pltpu.CompilerParams(needs_layout_passes=False),
        out_type=jax.ShapeDtypeStruct((DIM, SEQ, BATCH), jnp.float32),
        scratch_types=[
            pltpu.VMEM((VOCAB,), jnp.float32),
            pltpu.VMEM((8, BATCH), jnp.int32),
            pltpu.VMEM((8, BATCH), jnp.float32),
        ],
    )
    def gather_kernel(ids_hbm, table_hbm, out_hbm, row_v, ids_v, out_v):
        wid = lax.axis_index("s") * 2 + lax.axis_index("c")
        d_start = _DPW * wid + jnp.minimum(wid, _EXTRA)
        d_count = _DPW + jnp.where(wid < _EXTRA, 1, 0)

        def plane_body(k, carry):
            d = d_start + k
            pltpu.sync_copy(table_hbm.at[d], row_v)

            def band_body(band, carry2):
                pltpu.sync_copy(ids_hbm.at[pl.ds(band * 8, 8)], ids_v)

                @plsc.parallel_loop(0, 8 * BATCH, step=16, unroll=8)
                def gather_body(n):
                    r = n // BATCH
                    c = lax.rem(n, BATCH)
                    iv = ids_v[r, pl.ds(c, 16)]
                    out_v[r, pl.ds(c, 16)] = plsc.load_gather(row_v, [iv])

                pltpu.sync_copy(out_v, out_hbm.at[d, pl.ds(band * 8, 8)])
                return carry2

            lax.fori_loop(0, _NBANDS, band_body, 0)
            return carry

        lax.fori_loop(0, d_count, plane_body, 0)

    return gather_kernel


_gather = _make_gather()


def kernel(word_ids, emb_weight):
    out_p = _gather(word_ids.T, emb_weight.T)
    return out_p.transpose(2, 1, 0)
